# Initial kernel scaffold; baseline (speedup 1.0000x reference)
import jax
import jax.numpy as jnp
from jax.experimental import pallas as pl


def kernel(x, cluster_w, cluster_b, qkv_w, bais_w, bais_b):
    raise NotImplementedError("write your pallas kernel here")


# speedup vs baseline: 25.8318x; 25.8318x over previous
import jax
import jax.numpy as jnp
import numpy as np
from jax.experimental import pallas as pl

_B = 8
_C = 192
_H = 56
_NB = 4
_NH = 4
_HD = _C // _NH
_G = 4
_IPG = _C // _G
_HS = _H // 2
_LS = _HS * _HS
_L = _H * _H


def _fused_kernel(x4_ref, x_ref, wall_ref, cw_ref, cb_ref, a_ref, beff_ref,
                  bb_ref, o_ref):
    f32 = jnp.float32
    wall = wall_ref[...]

    m = jnp.dot(wall, x4_ref[0, 0], preferred_element_type=f32)
    pmax = m
    pmin = m
    for u in range(1, 4):
        m = jnp.dot(wall, x4_ref[0, u], preferred_element_type=f32)
        pmax = jnp.maximum(pmax, m)
        pmin = jnp.minimum(pmin, m)

    fs = []
    for i in range(_NB):
        acc = None
        for g in range(_G):
            cwg = cw_ref[i, g]
            wwt = jax.lax.dot_general(cwg, cwg, (((1,), (1,)), ((), ())),
                                      preferred_element_type=f32)
            rid = jax.lax.broadcasted_iota(jnp.int32, (_IPG, _IPG), 0)
            cid = jax.lax.broadcasted_iota(jnp.int32, (_IPG, _IPG), 1)
            dif = wwt - jnp.where(rid == cid, f32(1.0), f32(0.0))
            s = jnp.sum(dif * dif)
            acc = s if acc is None else acc + s
        fs.append(f32(1.0) - acc / f32(_G * _IPG * _IPG))

    hrow = jax.lax.broadcasted_iota(jnp.int32, (_NH, 64), 0)
    mcol = jax.lax.broadcasted_iota(jnp.int32, (_NH, 64), 1)
    smat = jnp.where(mcol // 16 == hrow, f32(1.0), f32(0.0))
    hcol = jax.lax.broadcasted_iota(jnp.int32, (64, _NH), 1)
    mrow = jax.lax.broadcasted_iota(jnp.int32, (64, _NH), 0)
    smat_t = jnp.where(mrow // 16 == hcol, f32(1.0), f32(0.0))

    K = jnp.zeros((_C, 1), f32)
    F = f32(1.0)
    for i in range(_NB):
        wi = wall[_C * i:_C * (i + 1), :]
        mixk = jnp.dot(wi, K, preferred_element_type=f32) + cb_ref[i]
        sel = jnp.where(F >= 0, pmax[_C * i:_C * (i + 1)],
                        pmin[_C * i:_C * (i + 1)])
        xds = F * sel + mixk
        prod = xds[0:64] * xds[64:128] * a_ref[i]
        scores = jnp.dot(smat, prod, preferred_element_type=f32)
        mx = jnp.max(scores, axis=1, keepdims=True)
        e = jnp.exp(scores - mx)
        attn = e / jnp.sum(e, axis=1, keepdims=True)
        attnb = jnp.dot(smat_t, attn, preferred_element_type=f32)
        wsum = jnp.sum(xds[128:192] * attnb, axis=1, keepdims=True)
        constv = jnp.dot(beff_ref[i], wsum, preferred_element_type=f32) \
            + bb_ref[i]
        K = constv + fs[i] * K
        F = F * fs[i]

    o_ref[0] = F * x_ref[0] + K


def kernel(x, cluster_w, cluster_b, qkv_w, bais_w, bais_b):
    f32 = jnp.float32
    x = x.astype(f32)
    x4 = x.reshape(_B, _C, _HS, 2, _HS, 2).transpose(0, 3, 5, 1, 2, 4) \
          .reshape(_B, 4, _C, _LS)
    xf = x.reshape(_B, _C, _L)

    eye_g = jnp.eye(_G, dtype=f32)
    wall = jnp.einsum('bgoi,gh->bgohi', cluster_w.astype(f32), eye_g) \
              .reshape(_NB * _C, _C)
    qw = qkv_w.astype(f32)
    aflat = ((qw[:, :_C] * qw[:, _C:2 * _C]).reshape(_NB, 64, 3).sum(-1)
             / np.sqrt(_HD).astype(np.float32))[..., None]
    beff = (bais_w.astype(f32) * qw[:, 2 * _C:][:, None, :]) \
        .reshape(_NB, _C, 64, 3).sum(-1)
    cb3 = cluster_b.astype(f32)[..., None]
    bb3 = bais_b.astype(f32)[..., None]

    out = pl.pallas_call(
        _fused_kernel,
        grid=(_B,),
        in_specs=[
            pl.BlockSpec((1, 4, _C, _LS), lambda b: (b, 0, 0, 0)),
            pl.BlockSpec((1, _C, _L), lambda b: (b, 0, 0)),
            pl.BlockSpec((_NB * _C, _C), lambda b: (0, 0)),
            pl.BlockSpec((_NB, _G, _IPG, _IPG), lambda b: (0, 0, 0, 0)),
            pl.BlockSpec((_NB, _C, 1), lambda b: (0, 0, 0)),
            pl.BlockSpec((_NB, 64, 1), lambda b: (0, 0, 0)),
            pl.BlockSpec((_NB, _C, 64), lambda b: (0, 0, 0)),
            pl.BlockSpec((_NB, _C, 1), lambda b: (0, 0, 0)),
        ],
        out_specs=pl.BlockSpec((1, _C, _L), lambda b: (b, 0, 0)),
        out_shape=jax.ShapeDtypeStruct((_B, _C, _L), f32),
    )(x4, xf, wall, cluster_w.astype(f32), cb3, aflat, beff, bb3)
    return out.reshape(_B, _C, _H, _H)


# speedup vs baseline: 40.6571x; 1.5739x over previous
import jax
import jax.numpy as jnp
import numpy as np
from jax.experimental import pallas as pl

_B = 8
_C = 192
_H = 56
_NB = 4
_NH = 4
_HD = _C // _NH
_G = 4
_IPG = _C // _G
_HS = _H // 2
_LS = _HS * _HS
_L = _H * _H
_NEG = -1e30


def _fused_kernel(x_ref, wall_ref, cw_ref, cb_ref, a_ref, beff_ref,
                  bb_ref, o_ref):
    f32 = jnp.float32
    xv = x_ref[0]
    wall = wall_ref[...]

    mt = jax.lax.dot_general(xv, wall, (((0,), (1,)), ((), ())),
                             preferred_element_type=f32)
    mt3 = mt.reshape(_HS, 2 * _H, _NB * _C)
    mv = jnp.maximum(mt3[:, :_H, :], mt3[:, _H:, :]).reshape(_HS * _H, _NB * _C)
    nv = jnp.minimum(mt3[:, :_H, :], mt3[:, _H:, :]).reshape(_HS * _H, _NB * _C)
    p2 = jnp.maximum(mv, jnp.roll(mv, -1, axis=0))
    n2 = jnp.minimum(nv, jnp.roll(nv, -1, axis=0))
    srow = jax.lax.broadcasted_iota(jnp.int32, (_HS * _H, 1), 0)
    even = (srow % 2) == 0

    fs = []
    for i in range(_NB):
        acc = None
        for g in range(_G):
            cwg = cw_ref[i, g]
            wwt = jax.lax.dot_general(cwg, cwg, (((1,), (1,)), ((), ())),
                                      preferred_element_type=f32)
            rid = jax.lax.broadcasted_iota(jnp.int32, (_IPG, _IPG), 0)
            cid = jax.lax.broadcasted_iota(jnp.int32, (_IPG, _IPG), 1)
            dif = wwt - jnp.where(rid == cid, f32(1.0), f32(0.0))
            s = jnp.sum(dif * dif)
            acc = s if acc is None else acc + s
        fs.append(f32(1.0) - acc / f32(_G * _IPG * _IPG))

    mrow = jax.lax.broadcasted_iota(jnp.int32, (64, _NH), 0)
    hcol = jax.lax.broadcasted_iota(jnp.int32, (64, _NH), 1)
    smat_h = jnp.where(mrow // 16 == hcol, f32(1.0), f32(0.0))

    K = jnp.zeros((_C, 1), f32)
    F = f32(1.0)
    for i in range(_NB):
        w_i = jnp.concatenate(
            [wall[64 * i:64 * i + 64],
             wall[256 + 64 * i:256 + 64 * i + 64],
             wall[512 + 64 * i:512 + 64 * i + 64]], axis=0)
        mixk = jax.lax.dot_general(K, w_i, (((0,), (1,)), ((), ())),
                                   preferred_element_type=f32) + cb_ref[i]
        pos = F >= 0
        x_u = F * jnp.where(pos, p2[:, 64 * i:64 * i + 64],
                            n2[:, 64 * i:64 * i + 64]) + mixk[:, 0:64]
        x_w = F * jnp.where(pos, p2[:, 256 + 64 * i:256 + 64 * i + 64],
                            n2[:, 256 + 64 * i:256 + 64 * i + 64]) \
            + mixk[:, 64:128]
        x_v = F * jnp.where(pos, p2[:, 512 + 64 * i:512 + 64 * i + 64],
                            n2[:, 512 + 64 * i:512 + 64 * i + 64]) \
            + mixk[:, 128:192]
        prod = x_u * x_w * a_ref[i]
        scores = jnp.dot(prod, smat_h, preferred_element_type=f32)
        sc = jnp.where(even, scores, f32(_NEG))
        mx = jnp.max(sc, axis=0, keepdims=True)
        e = jnp.exp(sc - mx)
        attn = e / jnp.sum(e, axis=0, keepdims=True)
        ws = jax.lax.dot_general(x_v, attn, (((0,), (0,)), ((), ())),
                                 preferred_element_type=f32)
        wsum = jnp.sum(ws * smat_h, axis=1, keepdims=True)
        constv = jnp.dot(beff_ref[i], wsum, preferred_element_type=f32) \
            + bb_ref[i]
        K = constv + fs[i] * K
        F = F * fs[i]

    o_ref[0] = F * xv + K


def kernel(x, cluster_w, cluster_b, qkv_w, bais_w, bais_b):
    f32 = jnp.float32
    x = x.astype(f32)
    xf = x.reshape(_B, _C, _L)

    eye_g = jnp.eye(_G, dtype=f32)
    wbd = jnp.einsum('bgoi,gh->bgohi', cluster_w.astype(f32), eye_g) \
             .reshape(_NB, _C, _C)
    wall = jnp.concatenate(
        [wbd[:, 0:64, :].reshape(_NB * 64, _C),
         wbd[:, 64:128, :].reshape(_NB * 64, _C),
         wbd[:, 128:192, :].reshape(_NB * 64, _C)], axis=0)
    qw = qkv_w.astype(f32)
    aflat = ((qw[:, :_C] * qw[:, _C:2 * _C]).reshape(_NB, 64, 3).sum(-1)
             / np.sqrt(_HD).astype(np.float32))[:, None, :]
    beff = (bais_w.astype(f32) * qw[:, 2 * _C:][:, None, :]) \
        .reshape(_NB, _C, 64, 3).sum(-1)
    cb2 = cluster_b.astype(f32)[:, None, :]
    bb3 = bais_b.astype(f32)[..., None]

    out = pl.pallas_call(
        _fused_kernel,
        grid=(_B,),
        in_specs=[
            pl.BlockSpec((1, _C, _L), lambda b: (b, 0, 0)),
            pl.BlockSpec((_NB * _C, _C), lambda b: (0, 0)),
            pl.BlockSpec((_NB, _G, _IPG, _IPG), lambda b: (0, 0, 0, 0)),
            pl.BlockSpec((_NB, 1, _C), lambda b: (0, 0, 0)),
            pl.BlockSpec((_NB, 1, 64), lambda b: (0, 0, 0)),
            pl.BlockSpec((_NB, _C, 64), lambda b: (0, 0, 0)),
            pl.BlockSpec((_NB, _C, 1), lambda b: (0, 0, 0)),
        ],
        out_specs=pl.BlockSpec((1, _C, _L), lambda b: (b, 0, 0)),
        out_shape=jax.ShapeDtypeStruct((_B, _C, _L), f32),
    )(xf, wall, cluster_w.astype(f32), cb2, aflat, beff, bb3)
    return out.reshape(_B, _C, _H, _H)
